# baseline (speedup 1.0000x reference)
import functools

import jax
import jax.numpy as jnp
from jax import lax
from jax.experimental import pallas as pl
from jax.experimental.pallas import tpu as pltpu
from jax.experimental.pallas import tpu_sc as plsc

N = 10000
E = 320000
H = 128
STEPS = 6
G = 16

NC = 2
NS = 16
NW = NC * NS

CH = 80
EPW = E // NW
NCHUNK = EPW // CH
ROWCHUNKS = N // CH
NB = 3
QC = 32
ZR = 640

@functools.cache
def _sc_kernels():
    mesh = plsc.VectorSubcoreMesh(
        core_axis_name="c", subcore_axis_name="s",
        num_cores=NC, num_subcores=NS)

    @functools.partial(
        pl.kernel,
        mesh=mesh,
        out_type=jax.ShapeDtypeStruct((N, H), jnp.float32),
        scratch_types=[
            pltpu.VMEM(((ROWCHUNKS + NW - 1) // NW, CH), jnp.int32),
            pltpu.VMEM(((ROWCHUNKS + NW - 1) // NW, CH, H), jnp.float32),
            [pltpu.SemaphoreType.DMA] * ((ROWCHUNKS + NW - 1) // NW),
            pltpu.SemaphoreType.DMA,
        ],
    )
    def embed_k(x_hbm, table_hbm, out_hbm, idx_v, rows_v, gsem, osem):
        wid = lax.axis_index("s") * NC + lax.axis_index("c")
        nj = (ROWCHUNKS + NW - 1) // NW
        for j in range(nj):
            c = wid + NW * j

            @pl.when(c < ROWCHUNKS)
            def _(j=j, c=c):
                pltpu.sync_copy(x_hbm.at[pl.ds(c * CH, CH)], idx_v.at[j])
                pltpu.async_copy(table_hbm.at[idx_v.at[j]], rows_v.at[j],
                                 gsem[j])

        for j in range(nj):
            c = wid + NW * j

            @pl.when(c < ROWCHUNKS)
            def _(j=j, c=c):
                pltpu.make_async_copy(table_hbm.at[idx_v.at[j]],
                                      rows_v.at[j], gsem[j]).wait()
                pltpu.async_copy(rows_v.at[j], out_hbm.at[pl.ds(c * CH, CH)],
                                 osem)

        for j in range(nj):
            c = wid + NW * j

            @pl.when(c < ROWCHUNKS)
            def _(j=j, c=c):
                pltpu.make_async_copy(rows_v.at[j],
                                      out_hbm.at[pl.ds(c * CH, CH)],
                                      osem).wait()

    @functools.partial(
        pl.kernel,
        mesh=mesh,
        out_type=jax.ShapeDtypeStruct((NC * N, H), jnp.float32),
        scratch_types=[
            pltpu.VMEM_SHARED((N, H), jnp.float32),
            pltpu.VMEM((QC, CH), jnp.int32),
            pltpu.VMEM((QC, CH), jnp.int32),
            pltpu.VMEM((NB, CH, H), jnp.float32),
            pltpu.SemaphoreType.DMA,
            [pltpu.SemaphoreType.DMA] * NB,
            [pltpu.SemaphoreType.DMA] * NB,
        ],
    )
    def edge_k(src_hbm, dst_hbm, m_hbm, zeros_hbm, out_hbm, agg_s, sidx,
               didx, rows, sem_i, gsem, ssem):
        cid = lax.axis_index("c")
        sid = lax.axis_index("s")
        wid = cid * NS + sid

        @pl.when(sid < NS - 1)
        def _():
            pltpu.sync_copy(zeros_hbm, agg_s.at[pl.ds(sid * ZR, ZR)])

        @pl.when(sid == NS - 1)
        def _():
            pltpu.sync_copy(zeros_hbm.at[pl.ds(0, N - (NS - 1) * ZR)],
                            agg_s.at[pl.ds((NS - 1) * ZR, N - (NS - 1) * ZR)])

        for q in range((NCHUNK + QC - 1) // QC):
            nq = min(QC, NCHUNK - q * QC)
            cp_s = pltpu.async_copy(src_hbm.at[wid, pl.ds(q * QC, nq)],
                                    sidx.at[pl.ds(0, nq)], sem_i)
            cp_d = pltpu.async_copy(dst_hbm.at[wid, pl.ds(q * QC, nq)],
                                    didx.at[pl.ds(0, nq)], sem_i)
            cp_s.wait()
            cp_d.wait()
            for b in range(NB):
                pltpu.async_copy(m_hbm.at[sidx.at[b]], rows.at[b], gsem[b])
            if q == 0:
                plsc.subcore_barrier()

            def group(g, carry):
                base = g * NB
                for b in range(NB):
                    c = base + b

                    @pl.when(c < nq)
                    def _(b=b, c=c):
                        pltpu.make_async_copy(m_hbm.at[sidx.at[c]],
                                              rows.at[b], gsem[b]).wait()
                        pltpu.async_copy(rows.at[b], agg_s.at[didx.at[c]],
                                         ssem[b], add=True)

                for b in range(NB):
                    c = base + b

                    @pl.when(c + NB < nq)
                    def _(b=b, c=c):
                        pltpu.make_async_copy(zeros_hbm.at[pl.ds(0, CH)],
                                              rows.at[b], ssem[b]).wait()
                        pltpu.async_copy(m_hbm.at[sidx.at[c + NB]],
                                         rows.at[b], gsem[b])

                return carry

            lax.fori_loop(0, (nq + NB - 1) // NB, group, 0)
            for b in range(NB):
                pltpu.make_async_copy(zeros_hbm.at[pl.ds(0, CH)], rows.at[b],
                                      ssem[b]).wait()

        plsc.subcore_barrier()

        @pl.when(sid < NS - 1)
        def _():
            pltpu.sync_copy(agg_s.at[pl.ds(sid * ZR, ZR)],
                            out_hbm.at[pl.ds(cid * N + sid * ZR, ZR)])

        @pl.when(sid == NS - 1)
        def _():
            pltpu.sync_copy(
                agg_s.at[pl.ds((NS - 1) * ZR, N - (NS - 1) * ZR)],
                out_hbm.at[pl.ds(cid * N + (NS - 1) * ZR, N - (NS - 1) * ZR)])

    return embed_k, edge_k


BR = 1000


def _mm_body(a_ref, b_ref, o_ref):
    o_ref[...] = jnp.dot(a_ref[...], b_ref[...],
                         preferred_element_type=jnp.float32)


def _mm(a, b):
    return pl.pallas_call(
        _mm_body,
        grid=(N // BR,),
        in_specs=[
            pl.BlockSpec((BR, a.shape[1]), lambda i: (i, 0)),
            pl.BlockSpec(b.shape, lambda i: (0, 0)),
        ],
        out_specs=pl.BlockSpec((BR, b.shape[1]), lambda i: (i, 0)),
        out_shape=jax.ShapeDtypeStruct((a.shape[0], b.shape[1]), jnp.float32),
    )(a, b)


def _gru_body(agg0_ref, agg1_ref, h_ref, wihT_ref, whhT_ref, bih_ref,
              bhh_ref, wnext_ref, hnew_ref, mnext_ref):
    agg = agg0_ref[...] + agg1_ref[...]
    h = h_ref[...]
    gi = jnp.dot(agg, wihT_ref[...], preferred_element_type=jnp.float32)
    gi = gi + bih_ref[...]
    gh = jnp.dot(h, whhT_ref[...], preferred_element_type=jnp.float32)
    gh = gh + bhh_ref[...]
    r = jax.nn.sigmoid(gi[:, :H] + gh[:, :H])
    z = jax.nn.sigmoid(gi[:, H:2 * H] + gh[:, H:2 * H])
    n = jnp.tanh(gi[:, 2 * H:] + r * gh[:, 2 * H:])
    hn = (1.0 - z) * n + z * h
    hnew_ref[...] = hn
    mnext_ref[...] = jnp.dot(hn, wnext_ref[...],
                             preferred_element_type=jnp.float32)


def _gru(agg0, agg1, h, wihT, whhT, bih, bhh, wnext):
    row = pl.BlockSpec((BR, H), lambda i: (i, 0))
    return pl.pallas_call(
        _gru_body,
        grid=(N // BR,),
        in_specs=[
            row, row, row,
            pl.BlockSpec((H, 3 * H), lambda i: (0, 0)),
            pl.BlockSpec((H, 3 * H), lambda i: (0, 0)),
            pl.BlockSpec((1, 3 * H), lambda i: (0, 0)),
            pl.BlockSpec((1, 3 * H), lambda i: (0, 0)),
            pl.BlockSpec((H, H), lambda i: (0, 0)),
        ],
        out_specs=[row, row],
        out_shape=[
            jax.ShapeDtypeStruct((N, H), jnp.float32),
            jax.ShapeDtypeStruct((N, H), jnp.float32),
        ],
    )(agg0, agg1, h, wihT, whhT, bih, bhh, wnext)


def _gru_pool_body(agg0_ref, agg1_ref, h_ref, wihT_ref, whhT_ref, bih_ref,
                   bhh_ref, batch_ref, lin1T_ref, lin1b_ref, outT_ref,
                   outb_ref, o_ref, sums_ref, counts_ref):
    i = pl.program_id(0)
    agg = agg0_ref[...] + agg1_ref[...]
    h = h_ref[...]
    gi = jnp.dot(agg, wihT_ref[...], preferred_element_type=jnp.float32)
    gi = gi + bih_ref[...]
    gh = jnp.dot(h, whhT_ref[...], preferred_element_type=jnp.float32)
    gh = gh + bhh_ref[...]
    r = jax.nn.sigmoid(gi[:, :H] + gh[:, :H])
    z = jax.nn.sigmoid(gi[:, H:2 * H] + gh[:, H:2 * H])
    n = jnp.tanh(gi[:, 2 * H:] + r * gh[:, 2 * H:])
    hr = jax.nn.relu((1.0 - z) * n + z * h)
    iota = lax.broadcasted_iota(jnp.int32, (G, BR), 0)
    bblk = batch_ref[0, 0, :]
    oh = (iota == bblk[None, :]).astype(jnp.float32)
    psum = jnp.dot(oh, hr, preferred_element_type=jnp.float32)
    pcnt = jnp.sum(oh, axis=1, keepdims=True)

    @pl.when(i == 0)
    def _():
        sums_ref[...] = psum
        counts_ref[...] = pcnt

    @pl.when(i > 0)
    def _():
        sums_ref[...] = sums_ref[...] + psum
        counts_ref[...] = counts_ref[...] + pcnt

    @pl.when(i == N // BR - 1)
    def _():
        pooled = sums_ref[...] / jnp.maximum(counts_ref[...], 1.0)
        h1 = jax.nn.relu(
            jnp.dot(pooled, lin1T_ref[...],
                    preferred_element_type=jnp.float32) + lin1b_ref[...])
        o_ref[...] = jnp.dot(h1, outT_ref[...],
                             preferred_element_type=jnp.float32) + outb_ref[...]


def _gru_pool(agg0, agg1, h, wihT, whhT, bih, bhh, batch2d, lin1T, lin1b,
              outT, outb):
    row = pl.BlockSpec((BR, H), lambda i: (i, 0))
    full = lambda s: pl.BlockSpec(s, lambda i: tuple(0 for _ in s))
    return pl.pallas_call(
        _gru_pool_body,
        grid=(N // BR,),
        in_specs=[
            row, row, row,
            full((H, 3 * H)), full((H, 3 * H)),
            full((1, 3 * H)), full((1, 3 * H)),
            pl.BlockSpec((1, 1, BR), lambda i: (i, 0, 0)),
            full((H, H)), full((1, H)), full((H, H)), full((1, H)),
        ],
        out_specs=full((G, H)),
        out_shape=jax.ShapeDtypeStruct((G, H), jnp.float32),
        scratch_shapes=[
            pltpu.VMEM((G, H), jnp.float32),
            pltpu.VMEM((G, 1), jnp.float32),
        ],
    )(agg0, agg1, h, wihT, whhT, bih, bhh, batch2d, lin1T, lin1b, outT, outb)


def kernel(x, edge_index, batch, emb_table, ggnn_w, w_ih, w_hh, b_ih, b_hh,
           lin1_w, lin1_b, out_w, out_b):
    src = edge_index[0].reshape(NW, NCHUNK, CH)
    dst = edge_index[1].reshape(NW, NCHUNK, CH)
    zeros_slab = jnp.zeros((ZR, H), jnp.float32)
    wihT = w_ih.T
    whhT = w_hh.T
    bih = b_ih.reshape(1, 3 * H)
    bhh = b_hh.reshape(1, 3 * H)

    lin1T = lin1_w.T
    lin1b = lin1_b.reshape(1, H)
    outT = jnp.zeros((H, H), jnp.float32).at[:, :2].set(out_w.T)
    outb = jnp.zeros((1, H), jnp.float32).at[0, :2].set(out_b)

    embed_k, edge_k = _sc_kernels()
    h = embed_k(x, emb_table)
    m = _mm(h, ggnn_w[0])
    for i in range(STEPS - 1):
        aggflat = edge_k(src, dst, m, zeros_slab)
        h, m = _gru(aggflat[:N], aggflat[N:], h, wihT, whhT, bih, bhh,
                    ggnn_w[i + 1])
    aggflat = edge_k(src, dst, m, zeros_slab)
    pooled_out = _gru_pool(aggflat[:N], aggflat[N:], h, wihT, whhT, bih, bhh,
                           batch.reshape(N // BR, 1, BR), lin1T, lin1b, outT,
                           outb)
    return pooled_out[:, :2]


# speedup vs baseline: 1.0174x; 1.0174x over previous
import functools

import jax
import jax.numpy as jnp
from jax import lax
from jax.experimental import pallas as pl
from jax.experimental.pallas import tpu as pltpu
from jax.experimental.pallas import tpu_sc as plsc

N = 10000
E = 320000
H = 128
STEPS = 6
G = 16

NC = 2
NS = 16
NW = NC * NS

CH = 80
EPW = E // NW
NCHUNK = EPW // CH
ROWCHUNKS = N // CH
NB = 3
QC = 32
ZR = 640

@functools.cache
def _sc_kernels():
    mesh = plsc.VectorSubcoreMesh(
        core_axis_name="c", subcore_axis_name="s",
        num_cores=NC, num_subcores=NS)

    @functools.partial(
        pl.kernel,
        mesh=mesh,
        out_type=jax.ShapeDtypeStruct((N, H), jnp.float32),
        scratch_types=[
            pltpu.VMEM(((ROWCHUNKS + NW - 1) // NW, CH), jnp.int32),
            pltpu.VMEM(((ROWCHUNKS + NW - 1) // NW, CH, H), jnp.float32),
            [pltpu.SemaphoreType.DMA] * ((ROWCHUNKS + NW - 1) // NW),
            pltpu.SemaphoreType.DMA,
        ],
    )
    def embed_k(x_hbm, table_hbm, out_hbm, idx_v, rows_v, gsem, osem):
        wid = lax.axis_index("s") * NC + lax.axis_index("c")
        nj = (ROWCHUNKS + NW - 1) // NW
        for j in range(nj):
            c = wid + NW * j

            @pl.when(c < ROWCHUNKS)
            def _(j=j, c=c):
                pltpu.sync_copy(x_hbm.at[pl.ds(c * CH, CH)], idx_v.at[j])
                pltpu.async_copy(table_hbm.at[idx_v.at[j]], rows_v.at[j],
                                 gsem[j])

        for j in range(nj):
            c = wid + NW * j

            @pl.when(c < ROWCHUNKS)
            def _(j=j, c=c):
                pltpu.make_async_copy(table_hbm.at[idx_v.at[j]],
                                      rows_v.at[j], gsem[j]).wait()
                pltpu.async_copy(rows_v.at[j], out_hbm.at[pl.ds(c * CH, CH)],
                                 osem)

        for j in range(nj):
            c = wid + NW * j

            @pl.when(c < ROWCHUNKS)
            def _(j=j, c=c):
                pltpu.make_async_copy(rows_v.at[j],
                                      out_hbm.at[pl.ds(c * CH, CH)],
                                      osem).wait()

    @functools.partial(
        pl.kernel,
        mesh=mesh,
        out_type=jax.ShapeDtypeStruct((NC * N, H), jnp.float32),
        scratch_types=[
            pltpu.VMEM_SHARED((N, H), jnp.float32),
            pltpu.VMEM((QC, CH), jnp.int32),
            pltpu.VMEM((QC, CH), jnp.int32),
            pltpu.VMEM((NB, CH, H), jnp.float32),
            pltpu.SemaphoreType.DMA,
            [pltpu.SemaphoreType.DMA] * NB,
            [pltpu.SemaphoreType.DMA] * NB,
        ],
    )
    def edge_k(src_hbm, dst_hbm, m_hbm, zeros_hbm, out_hbm, agg_s, sidx,
               didx, rows, sem_i, gsem, ssem):
        cid = lax.axis_index("c")
        sid = lax.axis_index("s")
        wid = cid * NS + sid

        @pl.when(sid < NS - 1)
        def _():
            pltpu.sync_copy(zeros_hbm, agg_s.at[pl.ds(sid * ZR, ZR)])

        @pl.when(sid == NS - 1)
        def _():
            pltpu.sync_copy(zeros_hbm.at[pl.ds(0, N - (NS - 1) * ZR)],
                            agg_s.at[pl.ds((NS - 1) * ZR, N - (NS - 1) * ZR)])

        for q in range((NCHUNK + QC - 1) // QC):
            nq = min(QC, NCHUNK - q * QC)
            cp_s = pltpu.async_copy(src_hbm.at[wid, pl.ds(q * QC, nq)],
                                    sidx.at[pl.ds(0, nq)], sem_i)
            cp_d = pltpu.async_copy(dst_hbm.at[wid, pl.ds(q * QC, nq)],
                                    didx.at[pl.ds(0, nq)], sem_i)
            cp_s.wait()
            cp_d.wait()
            for b in range(NB):
                pltpu.async_copy(m_hbm.at[sidx.at[b]], rows.at[b], gsem[b])
            if q == 0:
                plsc.subcore_barrier()

            def group(g, carry):
                base = g * NB
                for b in range(NB):
                    c = base + b

                    @pl.when(c < nq)
                    def _(b=b, c=c):
                        pltpu.make_async_copy(m_hbm.at[sidx.at[c]],
                                              rows.at[b], gsem[b]).wait()
                        pltpu.async_copy(rows.at[b], agg_s.at[didx.at[c]],
                                         ssem[b], add=True)

                for b in range(NB):
                    c = base + b

                    @pl.when(c + NB < nq)
                    def _(b=b, c=c):
                        pltpu.make_async_copy(zeros_hbm.at[pl.ds(0, CH)],
                                              rows.at[b], ssem[b]).wait()
                        pltpu.async_copy(m_hbm.at[sidx.at[c + NB]],
                                         rows.at[b], gsem[b])

                return carry

            lax.fori_loop(0, (nq + NB - 1) // NB, group, 0)
            for b in range(NB):
                pltpu.make_async_copy(zeros_hbm.at[pl.ds(0, CH)], rows.at[b],
                                      ssem[b]).wait()

        plsc.subcore_barrier()

        @pl.when(sid < NS - 1)
        def _():
            pltpu.sync_copy(agg_s.at[pl.ds(sid * ZR, ZR)],
                            out_hbm.at[pl.ds(cid * N + sid * ZR, ZR)])

        @pl.when(sid == NS - 1)
        def _():
            pltpu.sync_copy(
                agg_s.at[pl.ds((NS - 1) * ZR, N - (NS - 1) * ZR)],
                out_hbm.at[pl.ds(cid * N + (NS - 1) * ZR, N - (NS - 1) * ZR)])

    return embed_k, edge_k


BR = 2000


def _mm_body(a_ref, b_ref, o_ref):
    o_ref[...] = jnp.dot(a_ref[...], b_ref[...],
                         preferred_element_type=jnp.float32)


def _mm(a, b):
    return pl.pallas_call(
        _mm_body,
        grid=(N // BR,),
        in_specs=[
            pl.BlockSpec((BR, a.shape[1]), lambda i: (i, 0)),
            pl.BlockSpec(b.shape, lambda i: (0, 0)),
        ],
        out_specs=pl.BlockSpec((BR, b.shape[1]), lambda i: (i, 0)),
        out_shape=jax.ShapeDtypeStruct((a.shape[0], b.shape[1]), jnp.float32),
    )(a, b)


def _gru_body(agg0_ref, agg1_ref, h_ref, wihT_ref, whhT_ref, bih_ref,
              bhh_ref, wnext_ref, hnew_ref, mnext_ref):
    agg = agg0_ref[...] + agg1_ref[...]
    h = h_ref[...]
    gi = jnp.dot(agg, wihT_ref[...], preferred_element_type=jnp.float32)
    gi = gi + bih_ref[...]
    gh = jnp.dot(h, whhT_ref[...], preferred_element_type=jnp.float32)
    gh = gh + bhh_ref[...]
    r = jax.nn.sigmoid(gi[:, :H] + gh[:, :H])
    z = jax.nn.sigmoid(gi[:, H:2 * H] + gh[:, H:2 * H])
    n = jnp.tanh(gi[:, 2 * H:] + r * gh[:, 2 * H:])
    hn = (1.0 - z) * n + z * h
    hnew_ref[...] = hn
    mnext_ref[...] = jnp.dot(hn, wnext_ref[...],
                             preferred_element_type=jnp.float32)


def _gru(agg0, agg1, h, wihT, whhT, bih, bhh, wnext):
    row = pl.BlockSpec((BR, H), lambda i: (i, 0))
    return pl.pallas_call(
        _gru_body,
        grid=(N // BR,),
        in_specs=[
            row, row, row,
            pl.BlockSpec((H, 3 * H), lambda i: (0, 0)),
            pl.BlockSpec((H, 3 * H), lambda i: (0, 0)),
            pl.BlockSpec((1, 3 * H), lambda i: (0, 0)),
            pl.BlockSpec((1, 3 * H), lambda i: (0, 0)),
            pl.BlockSpec((H, H), lambda i: (0, 0)),
        ],
        out_specs=[row, row],
        out_shape=[
            jax.ShapeDtypeStruct((N, H), jnp.float32),
            jax.ShapeDtypeStruct((N, H), jnp.float32),
        ],
    )(agg0, agg1, h, wihT, whhT, bih, bhh, wnext)


def _gru_pool_body(agg0_ref, agg1_ref, h_ref, wihT_ref, whhT_ref, bih_ref,
                   bhh_ref, batch_ref, lin1T_ref, lin1b_ref, outT_ref,
                   outb_ref, o_ref, sums_ref, counts_ref):
    i = pl.program_id(0)
    agg = agg0_ref[...] + agg1_ref[...]
    h = h_ref[...]
    gi = jnp.dot(agg, wihT_ref[...], preferred_element_type=jnp.float32)
    gi = gi + bih_ref[...]
    gh = jnp.dot(h, whhT_ref[...], preferred_element_type=jnp.float32)
    gh = gh + bhh_ref[...]
    r = jax.nn.sigmoid(gi[:, :H] + gh[:, :H])
    z = jax.nn.sigmoid(gi[:, H:2 * H] + gh[:, H:2 * H])
    n = jnp.tanh(gi[:, 2 * H:] + r * gh[:, 2 * H:])
    hr = jax.nn.relu((1.0 - z) * n + z * h)
    iota = lax.broadcasted_iota(jnp.int32, (G, BR), 0)
    bblk = batch_ref[0, 0, :]
    oh = (iota == bblk[None, :]).astype(jnp.float32)
    psum = jnp.dot(oh, hr, preferred_element_type=jnp.float32)
    pcnt = jnp.sum(oh, axis=1, keepdims=True)

    @pl.when(i == 0)
    def _():
        sums_ref[...] = psum
        counts_ref[...] = pcnt

    @pl.when(i > 0)
    def _():
        sums_ref[...] = sums_ref[...] + psum
        counts_ref[...] = counts_ref[...] + pcnt

    @pl.when(i == N // BR - 1)
    def _():
        pooled = sums_ref[...] / jnp.maximum(counts_ref[...], 1.0)
        h1 = jax.nn.relu(
            jnp.dot(pooled, lin1T_ref[...],
                    preferred_element_type=jnp.float32) + lin1b_ref[...])
        o_ref[...] = jnp.dot(h1, outT_ref[...],
                             preferred_element_type=jnp.float32) + outb_ref[...]


def _gru_pool(agg0, agg1, h, wihT, whhT, bih, bhh, batch2d, lin1T, lin1b,
              outT, outb):
    row = pl.BlockSpec((BR, H), lambda i: (i, 0))
    full = lambda s: pl.BlockSpec(s, lambda i: tuple(0 for _ in s))
    return pl.pallas_call(
        _gru_pool_body,
        grid=(N // BR,),
        in_specs=[
            row, row, row,
            full((H, 3 * H)), full((H, 3 * H)),
            full((1, 3 * H)), full((1, 3 * H)),
            pl.BlockSpec((1, 1, BR), lambda i: (i, 0, 0)),
            full((H, H)), full((1, H)), full((H, H)), full((1, H)),
        ],
        out_specs=full((G, H)),
        out_shape=jax.ShapeDtypeStruct((G, H), jnp.float32),
        scratch_shapes=[
            pltpu.VMEM((G, H), jnp.float32),
            pltpu.VMEM((G, 1), jnp.float32),
        ],
    )(agg0, agg1, h, wihT, whhT, bih, bhh, batch2d, lin1T, lin1b, outT, outb)


def kernel(x, edge_index, batch, emb_table, ggnn_w, w_ih, w_hh, b_ih, b_hh,
           lin1_w, lin1_b, out_w, out_b):
    src = edge_index[0].reshape(NW, NCHUNK, CH)
    dst = edge_index[1].reshape(NW, NCHUNK, CH)
    zeros_slab = jnp.zeros((ZR, H), jnp.float32)
    wihT = w_ih.T
    whhT = w_hh.T
    bih = b_ih.reshape(1, 3 * H)
    bhh = b_hh.reshape(1, 3 * H)

    lin1T = lin1_w.T
    lin1b = lin1_b.reshape(1, H)
    outT = jnp.zeros((H, H), jnp.float32).at[:, :2].set(out_w.T)
    outb = jnp.zeros((1, H), jnp.float32).at[0, :2].set(out_b)

    embed_k, edge_k = _sc_kernels()
    h = embed_k(x, emb_table)
    m = _mm(h, ggnn_w[0])
    for i in range(STEPS - 1):
        aggflat = edge_k(src, dst, m, zeros_slab)
        h, m = _gru(aggflat[:N], aggflat[N:], h, wihT, whhT, bih, bhh,
                    ggnn_w[i + 1])
    aggflat = edge_k(src, dst, m, zeros_slab)
    pooled_out = _gru_pool(aggflat[:N], aggflat[N:], h, wihT, whhT, bih, bhh,
                           batch.reshape(N // BR, 1, BR), lin1T, lin1b, outT,
                           outb)
    return pooled_out[:, :2]
